# baseline (speedup 1.0000x reference)
import jax
import jax.numpy as jnp
from jax import lax
from jax.experimental import pallas as pl
from jax.experimental.pallas import tpu as pltpu
from jax.experimental.pallas import tpu_sc as plsc

_NUM_MAPPERS = 1000000
_EMBED_DIM = 64
_BATCH = 16384

_NC = 2
_NS = 16
_NW = _NC * _NS
_BPW = _BATCH // _NW
_FLIGHT = 16
_SLAB = 8
_MAX_BASE = _NUM_MAPPERS + 1 - _SLAB


def _lane_sum(v):
    lanes = lax.iota(jnp.int32, 16)
    dnums = lax.GatherDimensionNumbers(
        offset_dims=(), collapsed_slice_dims=(0,), start_index_map=(0,))
    for s in (8, 4, 2, 1):
        perm = lax.gather(v, (lanes ^ s)[:, None], dnums, (1,),
                          mode=lax.GatherScatterMode.PROMISE_IN_BOUNDS)
        v = v + perm
    return v


def _rsqrt(x):
    i = lax.bitcast_convert_type(x, jnp.int32)
    i = jnp.int32(0x5F3759DF) - lax.shift_right_arithmetic(i, 1)
    y = lax.bitcast_convert_type(i, jnp.float32)
    half = jnp.float32(0.5) * x
    for _ in range(3):
        y = y * (jnp.float32(1.5) - half * y * y)
    return y


def _embed_body(ids_hbm, tablet_hbm, gamma_hbm, beta_hbm, out_hbm,
                idx_v, slab_v, rows_v, gamma_v, beta_v, sem):
    wid = lax.axis_index("s") * _NC + lax.axis_index("c")
    base = wid * _BPW

    pltpu.sync_copy(ids_hbm.at[pl.ds(base, _BPW)], idx_v)
    pltpu.sync_copy(gamma_hbm, gamma_v)
    pltpu.sync_copy(beta_hbm, beta_v)

    for i in range(_BPW // 16):
        v = idx_v[pl.ds(i * 16, 16)]
        v = jnp.where(v == jnp.int32(-1), jnp.int32(_NUM_MAPPERS), v)
        v = jnp.minimum(jnp.maximum(v, jnp.int32(0)),
                        jnp.int32(_NUM_MAPPERS))
        idx_v[pl.ds(i * 16, 16)] = v

    g0 = gamma_v[pl.ds(0, 16)]
    g1 = gamma_v[pl.ds(16, 16)]
    g2 = gamma_v[pl.ds(32, 16)]
    g3 = gamma_v[pl.ds(48, 16)]
    b0 = beta_v[pl.ds(0, 16)]
    b1 = beta_v[pl.ds(16, 16)]
    b2 = beta_v[pl.ds(32, 16)]
    b3 = beta_v[pl.ds(48, 16)]

    inv_d = jnp.float32(1.0 / _EMBED_DIM)
    eps = jnp.float32(1e-5)
    lanes = lax.iota(jnp.int32, 16)

    def fire(ids16, k):
        rid = ids16[k]
        rbase = pl.multiple_of(
            (rid // jnp.int32(_SLAB)) * jnp.int32(_SLAB), _SLAB)
        pltpu.async_copy(tablet_hbm.at[:, pl.ds(rbase, _SLAB)],
                         slab_v.at[k], sem)
        return rid - rbase

    @plsc.parallel_loop(0, _BPW, step=_FLIGHT)
    def flight_loop(r0):
        ids16 = idx_v[pl.ds(r0, _FLIGHT)]
        offs = [fire(ids16, k) for k in range(_FLIGHT)]
        for k in range(_FLIGHT):
            pltpu.make_async_copy(tablet_hbm.at[:, pl.ds(0, _SLAB)],
                                  slab_v.at[k], sem).wait()
        for k in range(_FLIGHT):
            slab = slab_v.at[k]
            cidx = jnp.broadcast_to(offs[k], (16,))
            v0 = plsc.load_gather(slab, [lanes, cidx])
            v1 = plsc.load_gather(slab, [lanes + jnp.int32(16), cidx])
            v2 = plsc.load_gather(slab, [lanes + jnp.int32(32), cidx])
            v3 = plsc.load_gather(slab, [lanes + jnp.int32(48), cidx])
            tot = _lane_sum((v0 + v1) + (v2 + v3))
            mean = tot * inv_d
            t0 = v0 - mean
            t1 = v1 - mean
            t2 = v2 - mean
            t3 = v3 - mean
            sq = _lane_sum((t0 * t0 + t1 * t1) + (t2 * t2 + t3 * t3))
            rv = _rsqrt(sq * inv_d + eps)
            r = r0 + k
            rows_v[r, pl.ds(0, 16)] = t0 * rv * g0 + b0
            rows_v[r, pl.ds(16, 16)] = t1 * rv * g1 + b1
            rows_v[r, pl.ds(32, 16)] = t2 * rv * g2 + b2
            rows_v[r, pl.ds(48, 16)] = t3 * rv * g3 + b3

    pltpu.sync_copy(rows_v, out_hbm.at[pl.ds(base, _BPW), :])


@jax.jit
def _embed(mapper_ids, table, ln_gamma, ln_beta):
    mesh = plsc.VectorSubcoreMesh(core_axis_name="c", subcore_axis_name="s")
    f = pl.kernel(
        _embed_body,
        mesh=mesh,
        compiler_params=pltpu.CompilerParams(
            use_tc_tiling_on_sc=False, needs_layout_passes=False),
        out_type=jax.ShapeDtypeStruct((_BATCH, _EMBED_DIM), jnp.float32),
        scratch_types=[
            pltpu.VMEM((_BPW,), jnp.int32),
            pltpu.VMEM((_FLIGHT, _EMBED_DIM, _SLAB), jnp.float32),
            pltpu.VMEM((_BPW, _EMBED_DIM), jnp.float32),
            pltpu.VMEM((_EMBED_DIM,), jnp.float32),
            pltpu.VMEM((_EMBED_DIM,), jnp.float32),
            pltpu.SemaphoreType.DMA,
        ],
    )
    return f(mapper_ids, table.T, ln_gamma, ln_beta)


def kernel(mapper_ids, table, ln_gamma, ln_beta):
    return _embed(mapper_ids, table, ln_gamma, ln_beta)


# speedup vs baseline: 9.2648x; 9.2648x over previous
import jax
import jax.numpy as jnp
from jax import lax
from jax.experimental import pallas as pl
from jax.experimental.pallas import tpu as pltpu
from jax.experimental.pallas import tpu_sc as plsc

_NUM_MAPPERS = 1000000
_EMBED_DIM = 64
_PAD_DIM = 128
_BATCH = 16384

_NC = 2
_NS = 16
_NW = _NC * _NS
_BPW = _BATCH // _NW
_CHUNK = 128
_NJ = _BPW // _CHUNK


def _lane_sum(v):
    lanes = lax.iota(jnp.int32, 16)
    dnums = lax.GatherDimensionNumbers(
        offset_dims=(), collapsed_slice_dims=(0,), start_index_map=(0,))
    for s in (8, 4, 2, 1):
        perm = lax.gather(v, (lanes ^ s)[:, None], dnums, (1,),
                          mode=lax.GatherScatterMode.PROMISE_IN_BOUNDS)
        v = v + perm
    return v


def _rsqrt(x):
    i = lax.bitcast_convert_type(x, jnp.int32)
    i = jnp.int32(0x5F3759DF) - lax.shift_right_arithmetic(i, 1)
    y = lax.bitcast_convert_type(i, jnp.float32)
    half = jnp.float32(0.5) * x
    for _ in range(3):
        y = y * (jnp.float32(1.5) - half * y * y)
    return y


def _embed_body(ids_hbm, table_hbm, gamma_hbm, beta_hbm, out_hbm,
                idx2_v, rows_v, gamma_v, beta_v, sem):
    wid = lax.axis_index("s") * _NC + lax.axis_index("c")
    base = wid * _BPW

    for j in range(_NJ):
        pltpu.sync_copy(ids_hbm.at[pl.ds(base + j * _CHUNK, _CHUNK)],
                        idx2_v.at[j])
    pltpu.sync_copy(gamma_hbm, gamma_v)
    pltpu.sync_copy(beta_hbm, beta_v)

    for j in range(_NJ):
        for i in range(_CHUNK // 16):
            v = idx2_v[j, pl.ds(i * 16, 16)]
            v = jnp.where(v == jnp.int32(-1), jnp.int32(_NUM_MAPPERS), v)
            v = jnp.minimum(jnp.maximum(v, jnp.int32(0)),
                            jnp.int32(_NUM_MAPPERS))
            idx2_v[j, pl.ds(i * 16, 16)] = v

    copies = []
    for j in range(_NJ):
        copies.append(pltpu.async_copy(
            table_hbm.at[idx2_v.at[j]],
            rows_v.at[pl.ds(j * _CHUNK, _CHUNK), :],
            sem))
    for c in copies:
        c.wait()

    g0 = gamma_v[pl.ds(0, 16)]
    g1 = gamma_v[pl.ds(16, 16)]
    g2 = gamma_v[pl.ds(32, 16)]
    g3 = gamma_v[pl.ds(48, 16)]
    b0 = beta_v[pl.ds(0, 16)]
    b1 = beta_v[pl.ds(16, 16)]
    b2 = beta_v[pl.ds(32, 16)]
    b3 = beta_v[pl.ds(48, 16)]

    inv_d = jnp.float32(1.0 / _EMBED_DIM)
    eps = jnp.float32(1e-5)

    @plsc.parallel_loop(0, _BPW, unroll=4)
    def row_body(r):
        v0 = rows_v[r, pl.ds(0, 16)]
        v1 = rows_v[r, pl.ds(16, 16)]
        v2 = rows_v[r, pl.ds(32, 16)]
        v3 = rows_v[r, pl.ds(48, 16)]
        tot = _lane_sum((v0 + v1) + (v2 + v3))
        mean = tot * inv_d
        t0 = v0 - mean
        t1 = v1 - mean
        t2 = v2 - mean
        t3 = v3 - mean
        sq = _lane_sum((t0 * t0 + t1 * t1) + (t2 * t2 + t3 * t3))
        rv = _rsqrt(sq * inv_d + eps)
        rows_v[r, pl.ds(0, 16)] = t0 * rv * g0 + b0
        rows_v[r, pl.ds(16, 16)] = t1 * rv * g1 + b1
        rows_v[r, pl.ds(32, 16)] = t2 * rv * g2 + b2
        rows_v[r, pl.ds(48, 16)] = t3 * rv * g3 + b3

    pltpu.sync_copy(rows_v, out_hbm.at[pl.ds(base, _BPW), :])


@jax.jit
def _embed(mapper_ids, table, ln_gamma, ln_beta):
    mesh = plsc.VectorSubcoreMesh(core_axis_name="c", subcore_axis_name="s")
    f = pl.kernel(
        _embed_body,
        mesh=mesh,
        compiler_params=pltpu.CompilerParams(
            use_tc_tiling_on_sc=True, needs_layout_passes=False),
        out_type=jax.ShapeDtypeStruct((_BATCH, _PAD_DIM), jnp.float32),
        scratch_types=[
            pltpu.VMEM((_NJ, _CHUNK), jnp.int32),
            pltpu.VMEM((_BPW, _PAD_DIM), jnp.float32),
            pltpu.VMEM((_EMBED_DIM,), jnp.float32),
            pltpu.VMEM((_EMBED_DIM,), jnp.float32),
            pltpu.SemaphoreType.DMA,
        ],
    )
    table_p = jnp.pad(table, ((0, 0), (0, _PAD_DIM - _EMBED_DIM)))
    out_p = f(mapper_ids, table_p, ln_gamma, ln_beta)
    return out_p[:, :_EMBED_DIM]


def kernel(mapper_ids, table, ln_gamma, ln_beta):
    return _embed(mapper_ids, table, ln_gamma, ln_beta)
